# baseline (speedup 1.0000x reference)
import jax
import jax.numpy as jnp
from jax.experimental import pallas as pl
from jax.experimental.pallas import tpu as pltpu

IMG_N = 131
BATCH = 128
SB = 8
TOPK = 20
LO_D2 = (0.1 * (IMG_N * 0.5)) ** 2
HI_D2 = (0.6 * (IMG_N * 0.5)) ** 2
F32_BIG = 3.0e38


def _body(mean_ref, std_ref, tm_ref, u_ref, v_ref, pm_ref,
          pres_ref, wind_ref, out_ref, cw_ref, cm_ref):
    i = pl.program_id(0)
    m3 = mean_ref[3]
    s3 = std_ref[3]

    r = jax.lax.broadcasted_iota(jnp.int32, (SB, IMG_N, IMG_N), 1).astype(jnp.float32)
    c = jax.lax.broadcasted_iota(jnp.int32, (SB, IMG_N, IMG_N), 2).astype(jnp.float32)

    tm = tm_ref[:, 0] * s3 + m3
    tmin = jnp.min(tm, axis=(1, 2), keepdims=True)
    p = r * IMG_N + c
    fidx = jnp.min(jnp.where(tm == tmin, p, jnp.float32(3.0e7)),
                   axis=(1, 2), keepdims=True)
    cy = jnp.floor((fidx + 0.5) * (1.0 / IMG_N))
    cx = fidx - cy * IMG_N

    dx = c - cx
    dy = r - cy
    d2 = dx * dx + dy * dy
    ann = (d2 > LO_D2) & (d2 < HI_D2)

    u = u_ref[:, 0] * std_ref[0] + mean_ref[0]
    v = v_ref[:, 0] * std_ref[1] + mean_ref[1]
    cw_ref[...] = jnp.where(ann, u * u + v * v + 1e-6, jnp.float32(0.0))
    cm_ref[...] = pm_ref[:, 0] * s3 + m3

    def step(_, carry):
        accw, accp, remw, remp = carry
        cw = cw_ref[...]
        mw = jnp.max(cw, axis=(1, 2), keepdims=True)
        eqw = cw == mw
        cntw = jnp.sum(eqw.astype(jnp.float32), axis=(1, 2), keepdims=True)
        tkw = jnp.minimum(cntw, remw)
        accw = accw + tkw * jnp.sqrt(mw)
        remw = remw - tkw
        cw_ref[...] = jnp.where(eqw, jnp.float32(-1.0), cw)

        cm = cm_ref[...]
        mm = jnp.min(cm, axis=(1, 2), keepdims=True)
        eqm = cm == mm
        cntm = jnp.sum(eqm.astype(jnp.float32), axis=(1, 2), keepdims=True)
        tkm = jnp.minimum(cntm, remp)
        accp = accp + tkm * mm
        remp = remp - tkm
        cm_ref[...] = jnp.where(eqm, jnp.float32(F32_BIG), cm)
        return accw, accp, remw, remp

    zero = jnp.zeros((SB, 1, 1), jnp.float32)
    kk = jnp.full((SB, 1, 1), float(TOPK), jnp.float32)
    accw, accp, _, _ = jax.lax.fori_loop(
        0, TOPK, step, (zero, zero, kk, kk))

    pred_max_wind = accw * (1.0 / TOPK)
    pred_min_pres = accp * (1.0 / TOPK)

    contrib = (jnp.abs(pred_min_pres - pres_ref[...]) * 0.05
               + jnp.abs(pred_max_wind - 0.92 * wind_ref[...])) * (1.0 / BATCH)
    total = jnp.sum(contrib, axis=(0, 1, 2), keepdims=True)[:, :, 0]

    @pl.when(i == 0)
    def _():
        out_ref[...] = jnp.zeros((1, 1), jnp.float32)

    out_ref[...] += total


def kernel(pred_field, target_field, cma_pres_true, cma_wind_true, mean, std):
    def chan(c):
        return pl.BlockSpec((SB, 1, IMG_N, IMG_N), lambda i, c=c: (i, c, 0, 0))

    svec = pl.BlockSpec((SB, 1, 1), lambda i: (i, 0, 0))

    out = pl.pallas_call(
        _body,
        grid=(BATCH // SB,),
        in_specs=[
            pl.BlockSpec(memory_space=pltpu.SMEM),
            pl.BlockSpec(memory_space=pltpu.SMEM),
            chan(3), chan(0), chan(1), chan(3),
            svec, svec,
        ],
        out_specs=pl.BlockSpec((1, 1), lambda i: (0, 0)),
        out_shape=jax.ShapeDtypeStruct((1, 1), jnp.float32),
        scratch_shapes=[
            pltpu.VMEM((SB, IMG_N, IMG_N), jnp.float32),
            pltpu.VMEM((SB, IMG_N, IMG_N), jnp.float32),
        ],
    )(mean.reshape(4), std.reshape(4), target_field, pred_field, pred_field,
      pred_field, cma_pres_true.reshape(BATCH, 1, 1),
      cma_wind_true.reshape(BATCH, 1, 1))
    return out[0, 0]


# speedup vs baseline: 1.0190x; 1.0190x over previous
import jax
import jax.numpy as jnp
from jax.experimental import pallas as pl
from jax.experimental.pallas import tpu as pltpu

IMG_N = 131
BATCH = 128
SB = 16
TOPK = 20
LO_D2 = (0.1 * (IMG_N * 0.5)) ** 2
HI_D2 = (0.6 * (IMG_N * 0.5)) ** 2
F32_BIG = 3.0e38


def _body(mean_ref, std_ref, tm_ref, u_ref, v_ref, pm_ref,
          pres_ref, wind_ref, out_ref, cw_ref, cm_ref):
    i = pl.program_id(0)
    m3 = mean_ref[3]
    s3 = std_ref[3]

    r = jax.lax.broadcasted_iota(jnp.int32, (SB, IMG_N, IMG_N), 1).astype(jnp.float32)
    c = jax.lax.broadcasted_iota(jnp.int32, (SB, IMG_N, IMG_N), 2).astype(jnp.float32)

    tm = tm_ref[:, 0] * s3 + m3
    tmin = jnp.min(tm, axis=(1, 2), keepdims=True)
    p = r * IMG_N + c
    fidx = jnp.min(jnp.where(tm == tmin, p, jnp.float32(3.0e7)),
                   axis=(1, 2), keepdims=True)
    cy = jnp.floor((fidx + 0.5) * (1.0 / IMG_N))
    cx = fidx - cy * IMG_N

    dx = c - cx
    dy = r - cy
    d2 = dx * dx + dy * dy
    ann = (d2 > LO_D2) & (d2 < HI_D2)

    u = u_ref[:, 0] * std_ref[0] + mean_ref[0]
    v = v_ref[:, 0] * std_ref[1] + mean_ref[1]
    ws2 = jnp.where(ann, u * u + v * v + 1e-6, jnp.float32(0.0))
    pm = pm_ref[:, 0] * s3 + m3
    cw_ref[...] = ws2
    cm_ref[...] = pm

    mw0 = jnp.max(ws2, axis=(1, 2), keepdims=True)
    mm0 = jnp.min(pm, axis=(1, 2), keepdims=True)

    def step(_, carry):
        accw, accp, remw, remp, mw, mm = carry
        cw = cw_ref[...]
        eqw = cw == mw
        cntw = jnp.sum(eqw.astype(jnp.float32), axis=(1, 2), keepdims=True)
        tkw = jnp.minimum(cntw, remw)
        accw = accw + tkw * jnp.sqrt(mw)
        remw = remw - tkw
        nw = jnp.where(eqw, jnp.float32(-1.0), cw)
        cw_ref[...] = nw
        mw = jnp.max(nw, axis=(1, 2), keepdims=True)

        cm = cm_ref[...]
        eqm = cm == mm
        cntm = jnp.sum(eqm.astype(jnp.float32), axis=(1, 2), keepdims=True)
        tkm = jnp.minimum(cntm, remp)
        accp = accp + tkm * mm
        remp = remp - tkm
        nm = jnp.where(eqm, jnp.float32(F32_BIG), cm)
        cm_ref[...] = nm
        mm = jnp.min(nm, axis=(1, 2), keepdims=True)
        return accw, accp, remw, remp, mw, mm

    zero = jnp.zeros((SB, 1, 1), jnp.float32)
    kk = jnp.full((SB, 1, 1), float(TOPK), jnp.float32)
    accw, accp, _, _, _, _ = jax.lax.fori_loop(
        0, TOPK, step, (zero, zero, kk, kk, mw0, mm0), unroll=2)

    pred_max_wind = accw * (1.0 / TOPK)
    pred_min_pres = accp * (1.0 / TOPK)

    contrib = (jnp.abs(pred_min_pres - pres_ref[...]) * 0.05
               + jnp.abs(pred_max_wind - 0.92 * wind_ref[...])) * (1.0 / BATCH)
    total = jnp.sum(contrib, axis=(0, 1, 2), keepdims=True)[:, :, 0]

    @pl.when(i == 0)
    def _():
        out_ref[...] = jnp.zeros((1, 1), jnp.float32)

    out_ref[...] += total


def kernel(pred_field, target_field, cma_pres_true, cma_wind_true, mean, std):
    def chan(c):
        return pl.BlockSpec((SB, 1, IMG_N, IMG_N), lambda i, c=c: (i, c, 0, 0))

    svec = pl.BlockSpec((SB, 1, 1), lambda i: (i, 0, 0))

    out = pl.pallas_call(
        _body,
        grid=(BATCH // SB,),
        in_specs=[
            pl.BlockSpec(memory_space=pltpu.SMEM),
            pl.BlockSpec(memory_space=pltpu.SMEM),
            chan(3), chan(0), chan(1), chan(3),
            svec, svec,
        ],
        out_specs=pl.BlockSpec((1, 1), lambda i: (0, 0)),
        out_shape=jax.ShapeDtypeStruct((1, 1), jnp.float32),
        scratch_shapes=[
            pltpu.VMEM((SB, IMG_N, IMG_N), jnp.float32),
            pltpu.VMEM((SB, IMG_N, IMG_N), jnp.float32),
        ],
    )(mean.reshape(4), std.reshape(4), target_field, pred_field, pred_field,
      pred_field, cma_pres_true.reshape(BATCH, 1, 1),
      cma_wind_true.reshape(BATCH, 1, 1))
    return out[0, 0]


# speedup vs baseline: 1.2477x; 1.2245x over previous
import jax
import jax.numpy as jnp
from jax.experimental import pallas as pl
from jax.experimental.pallas import tpu as pltpu

IMG_N = 131
BATCH = 128
TOPK = 20
RB = 8
NRB = 17
LO_D2 = (0.1 * (IMG_N * 0.5)) ** 2
HI_D2 = (0.6 * (IMG_N * 0.5)) ** 2
F32_BIG = 3.0e38


def _center_body(tm_ref, fidx_ref):
    tm = tm_ref[:, 0]
    r = jax.lax.broadcasted_iota(jnp.int32, (8, IMG_N, IMG_N), 1).astype(jnp.float32)
    c = jax.lax.broadcasted_iota(jnp.int32, (8, IMG_N, IMG_N), 2).astype(jnp.float32)
    tmin = jnp.min(tm, axis=(1, 2), keepdims=True)
    p = r * IMG_N + c
    fidx = jnp.min(jnp.where(tm == tmin, p, jnp.float32(3.0e7)),
                   axis=(1, 2), keepdims=True)
    fidx_ref[...] = fidx[:, :, 0]


def _fields_body(mean_ref, std_ref, u_ref, v_ref, pm_ref, fidx_ref,
                 wst_ref, pmt_ref):
    i = pl.program_id(0)
    fidx = fidx_ref[...].reshape(BATCH, 1, 1)
    cy = jnp.floor((fidx + 0.5) * (1.0 / IMG_N))
    cx = fidx - cy * IMG_N

    r = jax.lax.broadcasted_iota(jnp.int32, (BATCH, RB, IMG_N), 1).astype(jnp.float32)
    c = jax.lax.broadcasted_iota(jnp.int32, (BATCH, RB, IMG_N), 2).astype(jnp.float32)
    row = r + (i * RB)
    dx = c - cx
    dy = row - cy
    d2 = dx * dx + dy * dy
    ann = (d2 > LO_D2) & (d2 < HI_D2) & (row < IMG_N)

    u = u_ref[:, 0] * std_ref[0] + mean_ref[0]
    v = v_ref[:, 0] * std_ref[1] + mean_ref[1]
    ws2 = jnp.where(ann, u * u + v * v + 1e-6, jnp.float32(0.0))
    pm = jnp.where(row < IMG_N, pm_ref[:, 0] * std_ref[3] + mean_ref[3],
                   jnp.float32(F32_BIG))

    for k in range(RB):
        wst_ref[k] = ws2[:, k, :].T
        pmt_ref[k] = pm[:, k, :].T


def _topw_body(wst_ref, maxw_ref, cw_ref):
    cw_ref[...] = wst_ref[...]
    mw0 = jnp.max(cw_ref[...], axis=(0, 1), keepdims=True)

    def step(_, carry):
        accw, remw, mw = carry
        cw = cw_ref[...]
        eqw = cw == mw
        cntw = jnp.sum(eqw.astype(jnp.float32), axis=(0, 1), keepdims=True)
        tkw = jnp.minimum(cntw, remw)
        accw = accw + tkw * jnp.sqrt(mw)
        remw = remw - tkw
        nw = jnp.where(eqw, jnp.float32(-1.0), cw)
        cw_ref[...] = nw
        return accw, remw, jnp.max(nw, axis=(0, 1), keepdims=True)

    zero = jnp.zeros((1, 1, BATCH), jnp.float32)
    kk = jnp.full((1, 1, BATCH), float(TOPK), jnp.float32)
    accw, _, _ = jax.lax.fori_loop(0, TOPK, step, (zero, kk, mw0), unroll=2)
    maxw_ref[...] = accw[0] * (1.0 / TOPK)


def _loss_body(pmt_ref, maxw_ref, pres_ref, wind_ref, out_ref, cm_ref):
    cm_ref[...] = pmt_ref[...]
    mm0 = jnp.min(cm_ref[...], axis=(0, 1), keepdims=True)

    def step(_, carry):
        accp, remp, mm = carry
        cm = cm_ref[...]
        eqm = cm == mm
        cntm = jnp.sum(eqm.astype(jnp.float32), axis=(0, 1), keepdims=True)
        tkm = jnp.minimum(cntm, remp)
        accp = accp + tkm * mm
        remp = remp - tkm
        nm = jnp.where(eqm, jnp.float32(F32_BIG), cm)
        cm_ref[...] = nm
        return accp, remp, jnp.min(nm, axis=(0, 1), keepdims=True)

    zero = jnp.zeros((1, 1, BATCH), jnp.float32)
    kk = jnp.full((1, 1, BATCH), float(TOPK), jnp.float32)
    accp, _, _ = jax.lax.fori_loop(0, TOPK, step, (zero, kk, mm0), unroll=2)
    pred_min_pres = accp[0] * (1.0 / TOPK)

    contrib = (jnp.abs(pred_min_pres - pres_ref[...]) * 0.05
               + jnp.abs(maxw_ref[...] - 0.92 * wind_ref[...])) * (1.0 / BATCH)
    out_ref[...] = jnp.sum(contrib, axis=(0, 1), keepdims=True)


def kernel(pred_field, target_field, cma_pres_true, cma_wind_true, mean, std):
    fidx = pl.pallas_call(
        _center_body,
        grid=(BATCH // 8,),
        in_specs=[pl.BlockSpec((8, 1, IMG_N, IMG_N), lambda i: (i, 3, 0, 0))],
        out_specs=pl.BlockSpec((8, 1), lambda i: (i, 0)),
        out_shape=jax.ShapeDtypeStruct((BATCH, 1), jnp.float32),
    )(target_field)

    def chan(c):
        return pl.BlockSpec((BATCH, 1, RB, IMG_N), lambda i, c=c: (0, c, i, 0))

    wst, pmt = pl.pallas_call(
        _fields_body,
        grid=(NRB,),
        in_specs=[
            pl.BlockSpec(memory_space=pltpu.SMEM),
            pl.BlockSpec(memory_space=pltpu.SMEM),
            chan(0), chan(1), chan(3),
            pl.BlockSpec((BATCH, 1), lambda i: (0, 0)),
        ],
        out_specs=[
            pl.BlockSpec((RB, IMG_N, BATCH), lambda i: (i, 0, 0)),
            pl.BlockSpec((RB, IMG_N, BATCH), lambda i: (i, 0, 0)),
        ],
        out_shape=[
            jax.ShapeDtypeStruct((NRB * RB, IMG_N, BATCH), jnp.float32),
            jax.ShapeDtypeStruct((NRB * RB, IMG_N, BATCH), jnp.float32),
        ],
    )(mean.reshape(4), std.reshape(4), pred_field, pred_field, pred_field,
      fidx)

    maxw = pl.pallas_call(
        _topw_body,
        in_specs=[pl.BlockSpec((NRB * RB, IMG_N, BATCH), lambda: (0, 0, 0))],
        out_specs=pl.BlockSpec((1, BATCH), lambda: (0, 0)),
        out_shape=jax.ShapeDtypeStruct((1, BATCH), jnp.float32),
        scratch_shapes=[pltpu.VMEM((NRB * RB, IMG_N, BATCH), jnp.float32)],
    )(wst)

    out = pl.pallas_call(
        _loss_body,
        in_specs=[
            pl.BlockSpec((NRB * RB, IMG_N, BATCH), lambda: (0, 0, 0)),
            pl.BlockSpec((1, BATCH), lambda: (0, 0)),
            pl.BlockSpec((1, BATCH), lambda: (0, 0)),
            pl.BlockSpec((1, BATCH), lambda: (0, 0)),
        ],
        out_specs=pl.BlockSpec((1, 1), lambda: (0, 0)),
        out_shape=jax.ShapeDtypeStruct((1, 1), jnp.float32),
        scratch_shapes=[pltpu.VMEM((NRB * RB, IMG_N, BATCH), jnp.float32)],
    )(pmt, maxw, cma_pres_true.reshape(1, BATCH), cma_wind_true.reshape(1, BATCH))
    return out[0, 0]


# speedup vs baseline: 1.3841x; 1.1093x over previous
import jax
import jax.numpy as jnp
from jax.experimental import pallas as pl
from jax.experimental.pallas import tpu as pltpu

IMG_N = 131
BATCH = 128
TOPK = 20
RB = 8
NRB = 17
LO_D2 = (0.1 * (IMG_N * 0.5)) ** 2
HI_D2 = (0.6 * (IMG_N * 0.5)) ** 2
F32_BIG = 3.0e38


def _center_body(tm_ref, fidx_ref):
    tm = tm_ref[:, 0]
    r = jax.lax.broadcasted_iota(jnp.int32, (8, IMG_N, IMG_N), 1).astype(jnp.float32)
    c = jax.lax.broadcasted_iota(jnp.int32, (8, IMG_N, IMG_N), 2).astype(jnp.float32)
    tmin = jnp.min(tm, axis=(1, 2), keepdims=True)
    p = r * IMG_N + c
    fidx = jnp.min(jnp.where(tm == tmin, p, jnp.float32(3.0e7)),
                   axis=(1, 2), keepdims=True)
    fidx_ref[...] = fidx[:, :, 0]


def _fields_body(mean_ref, std_ref, u_ref, v_ref, pm_ref, fidx_ref,
                 wst_ref, pmt_ref):
    i = pl.program_id(0)
    fidx = fidx_ref[...].reshape(BATCH, 1, 1)
    cy = jnp.floor((fidx + 0.5) * (1.0 / IMG_N))
    cx = fidx - cy * IMG_N

    r = jax.lax.broadcasted_iota(jnp.int32, (BATCH, RB, IMG_N), 1).astype(jnp.float32)
    c = jax.lax.broadcasted_iota(jnp.int32, (BATCH, RB, IMG_N), 2).astype(jnp.float32)
    row = r + (i * RB)
    dx = c - cx
    dy = row - cy
    d2 = dx * dx + dy * dy
    ann = (d2 > LO_D2) & (d2 < HI_D2) & (row < IMG_N)

    u = u_ref[:, 0] * std_ref[0] + mean_ref[0]
    v = v_ref[:, 0] * std_ref[1] + mean_ref[1]
    ws2 = jnp.where(ann, u * u + v * v + 1e-6, jnp.float32(0.0))
    pm = jnp.where(row < IMG_N, pm_ref[:, 0] * std_ref[3] + mean_ref[3],
                   jnp.float32(F32_BIG))

    for k in range(RB):
        wst_ref[k] = ws2[:, k, :].T
        pmt_ref[k] = pm[:, k, :].T


def _sel_body(wst_ref, pmt_ref, pres_ref, wind_ref, out_ref):
    mw0 = jnp.max(wst_ref[...], axis=(0, 1), keepdims=True)
    mm0 = jnp.min(pmt_ref[...], axis=(0, 1), keepdims=True)

    def step(_, carry):
        accw, accp, remw, remp, mw, mm = carry
        cw = wst_ref[...]
        eqw = cw == mw
        cntw = jnp.sum(eqw.astype(jnp.float32), axis=(0, 1), keepdims=True)
        tkw = jnp.minimum(cntw, remw)
        accw = accw + tkw * jnp.sqrt(mw)
        remw = remw - tkw
        nw = jnp.where(eqw, jnp.float32(-1.0), cw)
        wst_ref[...] = nw
        mw = jnp.max(nw, axis=(0, 1), keepdims=True)

        cm = pmt_ref[...]
        eqm = cm == mm
        cntm = jnp.sum(eqm.astype(jnp.float32), axis=(0, 1), keepdims=True)
        tkm = jnp.minimum(cntm, remp)
        accp = accp + tkm * mm
        remp = remp - tkm
        nm = jnp.where(eqm, jnp.float32(F32_BIG), cm)
        pmt_ref[...] = nm
        mm = jnp.min(nm, axis=(0, 1), keepdims=True)
        return accw, accp, remw, remp, mw, mm

    zero = jnp.zeros((1, 1, BATCH), jnp.float32)
    kk = jnp.full((1, 1, BATCH), float(TOPK), jnp.float32)
    accw, accp, _, _, _, _ = jax.lax.fori_loop(
        0, TOPK, step, (zero, zero, kk, kk, mw0, mm0), unroll=2)
    pred_max_wind = accw[0] * (1.0 / TOPK)
    pred_min_pres = accp[0] * (1.0 / TOPK)

    contrib = (jnp.abs(pred_min_pres - pres_ref[...]) * 0.05
               + jnp.abs(pred_max_wind - 0.92 * wind_ref[...])) * (1.0 / BATCH)
    out_ref[...] = jnp.sum(contrib, axis=(0, 1), keepdims=True)


def kernel(pred_field, target_field, cma_pres_true, cma_wind_true, mean, std):
    fidx = pl.pallas_call(
        _center_body,
        grid=(BATCH // 8,),
        in_specs=[pl.BlockSpec((8, 1, IMG_N, IMG_N), lambda i: (i, 3, 0, 0))],
        out_specs=pl.BlockSpec((8, 1), lambda i: (i, 0)),
        out_shape=jax.ShapeDtypeStruct((BATCH, 1), jnp.float32),
    )(target_field)

    def chan(c):
        return pl.BlockSpec((BATCH, 1, RB, IMG_N), lambda i, c=c: (0, c, i, 0))

    wst, pmt = pl.pallas_call(
        _fields_body,
        grid=(NRB,),
        in_specs=[
            pl.BlockSpec(memory_space=pltpu.SMEM),
            pl.BlockSpec(memory_space=pltpu.SMEM),
            chan(0), chan(1), chan(3),
            pl.BlockSpec((BATCH, 1), lambda i: (0, 0)),
        ],
        out_specs=[
            pl.BlockSpec((RB, IMG_N, BATCH), lambda i: (i, 0, 0)),
            pl.BlockSpec((RB, IMG_N, BATCH), lambda i: (i, 0, 0)),
        ],
        out_shape=[
            jax.ShapeDtypeStruct((NRB * RB, IMG_N, BATCH), jnp.float32),
            jax.ShapeDtypeStruct((NRB * RB, IMG_N, BATCH), jnp.float32),
        ],
    )(mean.reshape(4), std.reshape(4), pred_field, pred_field, pred_field,
      fidx)

    out = pl.pallas_call(
        _sel_body,
        in_specs=[
            pl.BlockSpec((NRB * RB, IMG_N, BATCH), lambda: (0, 0, 0)),
            pl.BlockSpec((NRB * RB, IMG_N, BATCH), lambda: (0, 0, 0)),
            pl.BlockSpec((1, BATCH), lambda: (0, 0)),
            pl.BlockSpec((1, BATCH), lambda: (0, 0)),
        ],
        out_specs=pl.BlockSpec((1, 1), lambda: (0, 0)),
        out_shape=jax.ShapeDtypeStruct((1, 1), jnp.float32),
    )(wst, pmt, cma_pres_true.reshape(1, BATCH), cma_wind_true.reshape(1, BATCH))
    return out[0, 0]
